# Initial kernel scaffold; baseline (speedup 1.0000x reference)
import jax
import jax.numpy as jnp
from jax.experimental import pallas as pl


def kernel(h, pos, batch_idx, params):
    raise NotImplementedError("write your pallas kernel here")


# speedup vs baseline: 130.5558x; 130.5558x over previous
import functools

import jax
import jax.numpy as jnp
from jax.experimental import pallas as pl
from jax.experimental.pallas import tpu as pltpu

HID = 128
CUTOFF = 10.0
PI = 3.14159
TD = 64
TS = 64


def _layer_kernel(nt, b_smem, p8_ref, h_ref, Wa, Wb, Gt, c2, b1, W2t, b2,
                  U1h, U1a, ub1, U2t, ub2, lng, lnb, w1, db1, out_ref):
    P = TS * TD
    D = pl.program_id(0)
    d0 = D * TD
    gmin = b_smem[d0]
    gmax = b_smem[d0 + TD - 1]

    sLo = jax.lax.while_loop(
        lambda t: jnp.logical_and(t > 0, b_smem[t * TS - 1] >= gmin),
        lambda t: t - 1, D)
    sHi = jax.lax.while_loop(
        lambda t: jnp.logical_and(t < nt - 1, b_smem[(t + 1) * TS] <= gmax),
        lambda t: t + 1, D)

    hd = h_ref[pl.ds(d0, TD), :]
    pd8 = p8_ref[pl.ds(d0, TD), :]
    Bh = jnp.dot(hd, Wb[:, :], preferred_element_type=jnp.float32)
    pdP = jnp.reshape(jnp.broadcast_to(pd8[None, :, :], (TS, TD, 8)), (P, 8))

    def body(t, acc):
        s0 = t * TS
        hs = h_ref[pl.ds(s0, TS), :]
        ps8 = p8_ref[pl.ds(s0, TS), :]
        Ah = jnp.dot(hs, Wa[:, :], preferred_element_type=jnp.float32)
        psP = jnp.reshape(jnp.broadcast_to(ps8[:, None, :], (TS, TD, 8)),
                          (P, 8))
        dif = psP[:, 0:3] - pdP[:, 0:3]
        dist = jnp.sqrt(jnp.sum(dif * dif, axis=1, keepdims=True))
        dist = jnp.maximum(dist, 1e-6)
        mask = ((psP[:, 3:4] == pdP[:, 3:4])
                & (psP[:, 4:5] != pdP[:, 4:5])
                & (dist < CUTOFF))
        cw = 0.5 * (1.0 + jnp.cos(PI * dist / CUTOFF))
        df1 = dist * w1[:, :] + db1[:, :]
        df1 = df1 * jax.nn.sigmoid(df1)
        dfc = (jnp.dot(df1, Gt[:, :], preferred_element_type=jnp.float32)
               + c2[:, :]) * cw
        base = jnp.reshape(Ah[:, None, :] + Bh[None, :, :], (P, HID))
        m1 = base + dfc + b1[:, :]
        m = m1 * jax.nn.sigmoid(m1)
        msg = jnp.dot(m, W2t[:, :], preferred_element_type=jnp.float32) \
            + b2[:, :]
        msg = msg * mask.astype(jnp.float32)
        return acc + jnp.sum(jnp.reshape(msg, (TS, TD, HID)), axis=0)

    acc = jax.lax.fori_loop(sLo, sHi + 1, body,
                            jnp.zeros((TD, HID), jnp.float32))

    u1 = (jnp.dot(hd, U1h[:, :], preferred_element_type=jnp.float32)
          + jnp.dot(acc, U1a[:, :], preferred_element_type=jnp.float32)
          + ub1[:, :])
    u = u1 * jax.nn.sigmoid(u1)
    hn = jnp.dot(u, U2t[:, :], preferred_element_type=jnp.float32) + ub2[:, :]
    x = hd + hn
    mu = jnp.mean(x, axis=1, keepdims=True)
    xc = x - mu
    var = jnp.mean(xc * xc, axis=1, keepdims=True)
    out_ref[:, :] = xc * jax.lax.rsqrt(var + 1e-5) * lng[:, :] + lnb[:, :]


def _prep_layer(p):
    W1 = p["msg_W1"]
    A, B, C = W1[:, :HID], W1[:, HID:2 * HID], W1[:, 2 * HID:]
    return (
        A.T, B.T,
        (C @ p["de_W2"]).T,
        (C @ p["de_b2"]).reshape(1, HID),
        p["msg_b1"].reshape(1, HID),
        p["msg_W2"].T,
        p["msg_b2"].reshape(1, HID),
        p["upd_W1"][:, :HID].T,
        p["upd_W1"][:, HID:].T,
        p["upd_b1"].reshape(1, HID),
        p["upd_W2"].T,
        p["upd_b2"].reshape(1, HID),
        p["ln_g"].reshape(1, HID),
        p["ln_b"].reshape(1, HID),
        p["de_W1"][:, 0].reshape(1, 16),
        p["de_b1"].reshape(1, 16),
    )


def _layer_call(h, p8, b32, weights):
    n = h.shape[0]
    nd = n // TD
    nt = n // TS
    full = lambda shape: pl.BlockSpec(shape, lambda i, b: (0,) * len(shape))
    wspecs = [full(w.shape) for w in weights]
    grid_spec = pltpu.PrefetchScalarGridSpec(
        num_scalar_prefetch=1,
        grid=(nd,),
        in_specs=[full((n, 8)), full((n, HID))] + wspecs,
        out_specs=pl.BlockSpec((TD, HID), lambda i, b: (i, 0)),
    )
    return pl.pallas_call(
        functools.partial(_layer_kernel, nt),
        grid_spec=grid_spec,
        out_shape=jax.ShapeDtypeStruct((n, HID), jnp.float32),
    )(b32, p8, h, *weights)


def kernel(h, pos, batch_idx, params):
    n = h.shape[0]
    b32 = batch_idx.astype(jnp.int32)
    p8 = jnp.concatenate(
        [pos.astype(jnp.float32),
         b32.astype(jnp.float32)[:, None],
         jnp.arange(n, dtype=jnp.float32)[:, None],
         jnp.zeros((n, 3), jnp.float32)], axis=1)
    for p in params["layers"]:
        h = _layer_call(h, p8, b32, _prep_layer(p))
    return h
